# initial kernel scaffold (unmeasured)
import jax
import jax.numpy as jnp
from jax import lax
from jax.experimental import pallas as pl
from jax.experimental.pallas import tpu as pltpu

N_DEV = 4
B = 2
SQ = 128
SKV = 128
HQ = 4
DH = 64
D_HEADS = HQ * DH
D_MODEL = 512
BLK = 64
BF16 = jnp.bfloat16
F32 = jnp.float32


def kernel(x, Wq, K_ext, V_ext, Wo):
    def body(x_ref, wq_ref, k_ref, v_ref, wo_ref, out_ref,
             kv_send, kv_recv, ar_stage, ar_recv,
             kv_send_sems, kv_recv_sem, ar_send_sems, ar_recv_sems):
        my_pos = lax.axis_index("i")

        barrier_sem = pltpu.get_barrier_semaphore()
        for off in range(1, N_DEV):
            peer = (my_pos + off) % N_DEV
            pl.semaphore_signal(barrier_sem, inc=1, device_id=(peer,),
                                device_id_type=pl.DeviceIdType.MESH)
        pl.semaphore_wait(barrier_sem, N_DEV - 1)

        def kv_scatter_rdma(j):
            return pltpu.make_async_remote_copy(
                src_ref=kv_send.at[j - 1],
                dst_ref=kv_recv,
                send_sem=kv_send_sems.at[j - 1],
                recv_sem=kv_recv_sem,
                device_id=(j,),
                device_id_type=pl.DeviceIdType.MESH,
            )

        @pl.when(my_pos == 0)
        def _():
            for b in range(B):
                for hl in range(HQ):
                    c = pl.ds(hl * DH, DH)
                    kv_recv[0, b, :, c] = k_ref[b, :, hl, :].astype(BF16)
                    kv_recv[1, b, :, c] = v_ref[b, :, hl, :].astype(BF16)
            for j in range(1, N_DEV):
                for b in range(B):
                    for hl in range(HQ):
                        h = j * HQ + hl
                        c = pl.ds(hl * DH, DH)
                        kv_send[j - 1, 0, b, :, c] = k_ref[b, :, h, :].astype(BF16)
                        kv_send[j - 1, 1, b, :, c] = v_ref[b, :, h, :].astype(BF16)
            for j in range(1, N_DEV):
                kv_scatter_rdma(j).start()

        wq = wq_ref[:].astype(BF16)
        wo = wo_ref[:].astype(BF16)
        q = []
        for b in range(B):
            xb = x_ref[b].astype(BF16)
            q.append(lax.dot(xb, wq, preferred_element_type=BF16))

        @pl.when(my_pos != 0)
        def _():
            kv_scatter_rdma(1).wait_recv()

        qb = lax.broadcasted_iota(jnp.int32, (SQ, SKV), 0) // BLK
        kb = lax.broadcasted_iota(jnp.int32, (SQ, SKV), 1) // BLK
        mask = kb <= qb

        acc = []
        for b in range(B):
            acc_b = None
            for hl in range(HQ):
                c = pl.ds(hl * DH, DH)
                qh = q[b][:, c]
                kh = kv_recv[0, b, :, c]
                vh = kv_recv[1, b, :, c]
                s = lax.dot_general(qh, kh, (((1,), (1,)), ((), ())),
                                    preferred_element_type=F32)
                s = jnp.where(mask, s * 0.125, -1e9)
                m = jnp.max(s, axis=1, keepdims=True)
                w = jnp.exp(s - m)
                w = w / jnp.sum(w, axis=1, keepdims=True)
                ctx = lax.dot(w.astype(BF16), vh,
                              preferred_element_type=F32)
                contrib = lax.dot(ctx.astype(BF16), wo[c, :],
                                  preferred_element_type=F32)
                acc_b = contrib if acc_b is None else acc_b + contrib
            acc.append(acc_b)

        @pl.when(my_pos == 0)
        def _():
            for j in range(1, N_DEV):
                kv_scatter_rdma(j).wait_send()

        for b in range(B):
            ar_stage[b] = acc[b].astype(BF16)
        ar_rdmas = []
        for off in range(1, N_DEV):
            dest = (my_pos + off) % N_DEV
            rdma = pltpu.make_async_remote_copy(
                src_ref=ar_stage,
                dst_ref=ar_recv.at[off - 1],
                send_sem=ar_send_sems.at[off - 1],
                recv_sem=ar_recv_sems.at[off - 1],
                device_id=(dest,),
                device_id_type=pl.DeviceIdType.MESH,
            )
            rdma.start()
            ar_rdmas.append(rdma)
        for rdma in ar_rdmas:
            rdma.wait()

        for b in range(B):
            total = acc[b]
            for off in range(1, N_DEV):
                total = total + ar_recv[off - 1, b].astype(F32)
            out_ref[b] = total

    return pl.pallas_call(
        body,
        out_shape=jax.ShapeDtypeStruct((B, SQ, D_MODEL), F32),
        in_specs=[pl.BlockSpec(memory_space=pltpu.VMEM)] * 5,
        out_specs=pl.BlockSpec(memory_space=pltpu.VMEM),
        scratch_shapes=[
            pltpu.VMEM((N_DEV - 1, 2, B, SKV, D_HEADS), BF16),
            pltpu.VMEM((2, B, SKV, D_HEADS), BF16),
            pltpu.VMEM((B, SQ, D_MODEL), BF16),
            pltpu.VMEM((N_DEV - 1, B, SQ, D_MODEL), BF16),
            pltpu.SemaphoreType.DMA((N_DEV - 1,)),
            pltpu.SemaphoreType.DMA,
            pltpu.SemaphoreType.DMA((N_DEV - 1,)),
            pltpu.SemaphoreType.DMA((N_DEV - 1,)),
        ],
        compiler_params=pltpu.CompilerParams(collective_id=0),
    )(x, Wq, K_ext, V_ext, Wo)


# baseline (device time: 28885 ns/iter reference)
import jax
import jax.numpy as jnp
from jax import lax
from jax.experimental import pallas as pl
from jax.experimental.pallas import tpu as pltpu

N_DEV = 4
B = 2
SQ = 128
SKV = 128
HQ = 4
DH = 64
D_HEADS = HQ * DH
D_MODEL = 512
BLK = 64
BF16 = jnp.bfloat16
F32 = jnp.float32


def kernel(x, Wq, K_ext, V_ext, Wo):
    def body(x_ref, wq_ref, k_ref, v_ref, wo_ref, out_ref,
             kv_send, kv_recv, ar_stage, ar_recv,
             kv_send_sems, kv_recv_sem, ar_send_sems, ar_recv_sems):
        my_pos = lax.axis_index("i")

        barrier_sem = pltpu.get_barrier_semaphore()
        for off in range(1, N_DEV):
            peer = (my_pos + off) % N_DEV
            pl.semaphore_signal(barrier_sem, inc=1, device_id=(peer,),
                                device_id_type=pl.DeviceIdType.MESH)
        pl.semaphore_wait(barrier_sem, N_DEV - 1)

        def kv_scatter_rdma(j):
            return pltpu.make_async_remote_copy(
                src_ref=kv_send.at[j - 1],
                dst_ref=kv_recv,
                send_sem=kv_send_sems.at[j - 1],
                recv_sem=kv_recv_sem,
                device_id=(j,),
                device_id_type=pl.DeviceIdType.MESH,
            )

        @pl.when(my_pos == 0)
        def _():
            for b in range(B):
                for hl in range(HQ):
                    c = slice(hl * DH, (hl + 1) * DH)
                    kv_recv[0, b, :, c] = k_ref[b, :, hl, :].astype(BF16)
                    kv_recv[1, b, :, c] = v_ref[b, :, hl, :].astype(BF16)
            for j in range(1, N_DEV):
                for b in range(B):
                    for hl in range(HQ):
                        h = j * HQ + hl
                        c = slice(hl * DH, (hl + 1) * DH)
                        kv_send[j - 1, 0, b, :, c] = k_ref[b, :, h, :].astype(BF16)
                        kv_send[j - 1, 1, b, :, c] = v_ref[b, :, h, :].astype(BF16)
            for j in range(1, N_DEV):
                kv_scatter_rdma(j).start()

        wq = wq_ref[:].astype(BF16)
        wo = wo_ref[:].astype(BF16)
        q = []
        for b in range(B):
            xb = x_ref[b].astype(BF16)
            q.append(lax.dot(xb, wq,
                             preferred_element_type=F32).astype(BF16))

        @pl.when(my_pos != 0)
        def _():
            kv_scatter_rdma(1).wait_recv()

        qb = lax.broadcasted_iota(jnp.int32, (SQ, SKV), 0) // BLK
        kb = lax.broadcasted_iota(jnp.int32, (SQ, SKV), 1) // BLK
        mask = kb <= qb

        acc = []
        for b in range(B):
            acc_b = None
            for hl in range(HQ):
                c = slice(hl * DH, (hl + 1) * DH)
                qh = q[b][:, c]
                kh = kv_recv[0, b, :, c]
                vh = kv_recv[1, b, :, c]
                s = lax.dot_general(qh, kh, (((1,), (1,)), ((), ())),
                                    preferred_element_type=F32)
                s = jnp.where(mask, s * 0.125, -1e9)
                m = jnp.max(s, axis=1, keepdims=True)
                w = jnp.exp(s - m)
                w = w / jnp.sum(w, axis=1, keepdims=True)
                ctx = lax.dot(w.astype(BF16), vh,
                              preferred_element_type=F32)
                contrib = lax.dot(ctx.astype(BF16), wo[c, :],
                                  preferred_element_type=F32)
                acc_b = contrib if acc_b is None else acc_b + contrib
            acc.append(acc_b)

        @pl.when(my_pos == 0)
        def _():
            for j in range(1, N_DEV):
                kv_scatter_rdma(j).wait_send()

        for b in range(B):
            ar_stage[b] = acc[b].astype(BF16)
        ar_rdmas = []
        for off in range(1, N_DEV):
            dest = (my_pos + off) % N_DEV
            rdma = pltpu.make_async_remote_copy(
                src_ref=ar_stage,
                dst_ref=ar_recv.at[off - 1],
                send_sem=ar_send_sems.at[off - 1],
                recv_sem=ar_recv_sems.at[off - 1],
                device_id=(dest,),
                device_id_type=pl.DeviceIdType.MESH,
            )
            rdma.start()
            ar_rdmas.append(rdma)
        for rdma in ar_rdmas:
            rdma.wait()

        for b in range(B):
            total = acc[b]
            for off in range(1, N_DEV):
                total = total + ar_recv[off - 1, b].astype(F32)
            out_ref[b] = total

    return pl.pallas_call(
        body,
        out_shape=jax.ShapeDtypeStruct((B, SQ, D_MODEL), F32),
        in_specs=[pl.BlockSpec(memory_space=pltpu.VMEM)] * 5,
        out_specs=pl.BlockSpec(memory_space=pltpu.VMEM),
        scratch_shapes=[
            pltpu.VMEM((N_DEV - 1, 2, B, SKV, D_HEADS), BF16),
            pltpu.VMEM((2, B, SKV, D_HEADS), BF16),
            pltpu.VMEM((B, SQ, D_MODEL), BF16),
            pltpu.VMEM((N_DEV - 1, B, SQ, D_MODEL), BF16),
            pltpu.SemaphoreType.DMA((N_DEV - 1,)),
            pltpu.SemaphoreType.DMA,
            pltpu.SemaphoreType.DMA((N_DEV - 1,)),
            pltpu.SemaphoreType.DMA((N_DEV - 1,)),
        ],
        compiler_params=pltpu.CompilerParams(collective_id=0),
    )(x, Wq, K_ext, V_ext, Wo)


# device time: 23862 ns/iter; 1.2105x vs baseline; 1.2105x over previous
import jax
import jax.numpy as jnp
from jax import lax
from jax.experimental import pallas as pl
from jax.experimental.pallas import tpu as pltpu

N_DEV = 4
B = 2
SQ = 128
SKV = 128
HQ = 4
DH = 64
HQ_ALL = 16
D_HEADS = HQ * DH
D_MODEL = 512
BLK = 64
BF16 = jnp.bfloat16
F32 = jnp.float32


def kernel(x, Wq, K_ext, V_ext, Wo):
    def body(x_ref, wq_ref, k_ref, v_ref, wo_ref, out_ref,
             kv_f32, kv_send, kv_recv, ar_stage, ar_recv,
             local_sems, kv_send_sems, kv_recv_sem, ar_send_sems, ar_recv_sems):
        my_pos = lax.axis_index("i")

        cp_k = pltpu.make_async_copy(k_ref, kv_f32.at[0], local_sems.at[0])
        cp_v = pltpu.make_async_copy(v_ref, kv_f32.at[1], local_sems.at[1])

        @pl.when(my_pos == 0)
        def _():
            cp_k.start()
            cp_v.start()

        barrier_sem = pltpu.get_barrier_semaphore()
        for off in range(1, N_DEV):
            peer = (my_pos + off) % N_DEV
            pl.semaphore_signal(barrier_sem, inc=1, device_id=(peer,),
                                device_id_type=pl.DeviceIdType.MESH)
        pl.semaphore_wait(barrier_sem, N_DEV - 1)

        def kv_scatter_rdma(j):
            return pltpu.make_async_remote_copy(
                src_ref=kv_send.at[j - 1],
                dst_ref=kv_recv,
                send_sem=kv_send_sems.at[j - 1],
                recv_sem=kv_recv_sem,
                device_id=(j,),
                device_id_type=pl.DeviceIdType.MESH,
            )

        @pl.when(my_pos == 0)
        def _():
            cp_k.wait()
            cp_v.wait()
            for t in range(2):
                for b in range(B):
                    flat = jnp.reshape(kv_f32[t, b].astype(BF16),
                                       (SKV, HQ_ALL * DH))
                    kv_recv[t, b] = flat[:, 0:D_HEADS]
                    for j in range(1, N_DEV):
                        kv_send[j - 1, t, b] = (
                            flat[:, j * D_HEADS:(j + 1) * D_HEADS])
            for j in range(1, N_DEV):
                kv_scatter_rdma(j).start()

        wq = wq_ref[:].astype(BF16)
        wo = wo_ref[:].astype(BF16)
        q = []
        for b in range(B):
            xb = x_ref[b].astype(BF16)
            q.append(lax.dot(xb, wq,
                             preferred_element_type=F32).astype(BF16))

        @pl.when(my_pos != 0)
        def _():
            kv_scatter_rdma(1).wait_recv()

        qb = lax.broadcasted_iota(jnp.int32, (SQ, SKV), 0) // BLK
        kb = lax.broadcasted_iota(jnp.int32, (SQ, SKV), 1) // BLK
        mask = kb <= qb

        acc = []
        ar_rdmas = []
        for b in range(B):
            acc_b = None
            for hl in range(HQ):
                c = slice(hl * DH, (hl + 1) * DH)
                qh = q[b][:, c]
                kh = kv_recv[0, b, :, c]
                vh = kv_recv[1, b, :, c]
                s = lax.dot_general(qh, kh, (((1,), (1,)), ((), ())),
                                    preferred_element_type=F32)
                s = jnp.where(mask, s * 0.125, -1e9)
                m = jnp.max(s, axis=1, keepdims=True)
                w = jnp.exp(s - m)
                w = w / jnp.sum(w, axis=1, keepdims=True)
                ctx = lax.dot(w.astype(BF16), vh,
                              preferred_element_type=F32)
                contrib = lax.dot(ctx.astype(BF16), wo[c, :],
                                  preferred_element_type=F32)
                acc_b = contrib if acc_b is None else acc_b + contrib
            acc.append(acc_b)
            ar_stage[b] = acc_b.astype(BF16)
            for off in range(1, N_DEV):
                dest = (my_pos + off) % N_DEV
                rdma = pltpu.make_async_remote_copy(
                    src_ref=ar_stage.at[b],
                    dst_ref=ar_recv.at[off - 1, b],
                    send_sem=ar_send_sems.at[off - 1, b],
                    recv_sem=ar_recv_sems.at[off - 1, b],
                    device_id=(dest,),
                    device_id_type=pl.DeviceIdType.MESH,
                )
                rdma.start()
                ar_rdmas.append(rdma)

        @pl.when(my_pos == 0)
        def _():
            for j in range(1, N_DEV):
                kv_scatter_rdma(j).wait_send()

        for rdma in ar_rdmas:
            rdma.wait()

        for b in range(B):
            total = acc[b]
            for off in range(1, N_DEV):
                total = total + ar_recv[off - 1, b].astype(F32)
            out_ref[b] = total

    return pl.pallas_call(
        body,
        out_shape=jax.ShapeDtypeStruct((B, SQ, D_MODEL), F32),
        in_specs=[
            pl.BlockSpec(memory_space=pltpu.VMEM),
            pl.BlockSpec(memory_space=pltpu.VMEM),
            pl.BlockSpec(memory_space=pl.ANY),
            pl.BlockSpec(memory_space=pl.ANY),
            pl.BlockSpec(memory_space=pltpu.VMEM),
        ],
        out_specs=pl.BlockSpec(memory_space=pltpu.VMEM),
        scratch_shapes=[
            pltpu.VMEM((2, B, SKV, HQ_ALL, DH), F32),
            pltpu.VMEM((N_DEV - 1, 2, B, SKV, D_HEADS), BF16),
            pltpu.VMEM((2, B, SKV, D_HEADS), BF16),
            pltpu.VMEM((B, SQ, D_MODEL), BF16),
            pltpu.VMEM((N_DEV - 1, B, SQ, D_MODEL), BF16),
            pltpu.SemaphoreType.DMA((2,)),
            pltpu.SemaphoreType.DMA((N_DEV - 1,)),
            pltpu.SemaphoreType.DMA,
            pltpu.SemaphoreType.DMA((N_DEV - 1, B)),
            pltpu.SemaphoreType.DMA((N_DEV - 1, B)),
        ],
        compiler_params=pltpu.CompilerParams(collective_id=0),
    )(x, Wq, K_ext, V_ext, Wo)


# device time: 21240 ns/iter; 1.3599x vs baseline; 1.1234x over previous
import jax
import jax.numpy as jnp
from jax import lax
from jax.experimental import pallas as pl
from jax.experimental.pallas import tpu as pltpu

N_DEV = 4
B = 2
SQ = 128
SKV = 128
HQ = 4
DH = 64
HQ_ALL = 16
D_HEADS = HQ * DH
D_MODEL = 512
BLK = 64
BF16 = jnp.bfloat16
F32 = jnp.float32

CHUNKS = [(0, 0), (1, 0), (0, 1), (1, 1)]


def kernel(x, Wq, K_ext, V_ext, Wo):
    def body(x_ref, wq_ref, k_ref, v_ref, wo_ref, out_ref,
             kv_f32, kv_send, kv_recv, ar_stage, ar_recv,
             local_sems, kv_send_sems, kv_recv_sems, ar_send_sems, ar_recv_sems):
        my_pos = lax.axis_index("i")

        tv_ref = [k_ref, v_ref]
        cps = {
            (t, b): pltpu.make_async_copy(
                tv_ref[t].at[b], kv_f32.at[t, b], local_sems.at[t, b])
            for (t, b) in CHUNKS
        }

        @pl.when(my_pos == 0)
        def _():
            for tb in CHUNKS:
                cps[tb].start()

        barrier_sem = pltpu.get_barrier_semaphore()
        for off in range(1, N_DEV):
            peer = (my_pos + off) % N_DEV
            pl.semaphore_signal(barrier_sem, inc=1, device_id=(peer,),
                                device_id_type=pl.DeviceIdType.MESH)
        pl.semaphore_wait(barrier_sem, N_DEV - 1)

        wq = wq_ref[:].astype(BF16)
        wo = wo_ref[:].astype(BF16)
        q = []
        for b in range(B):
            xb = x_ref[b].astype(BF16)
            q.append(lax.dot(xb, wq,
                             preferred_element_type=F32).astype(BF16))

        def kv_scatter_rdma(j, t, b):
            return pltpu.make_async_remote_copy(
                src_ref=kv_send.at[j - 1, t, b],
                dst_ref=kv_recv.at[t, b],
                send_sem=kv_send_sems.at[j - 1, t, b],
                recv_sem=kv_recv_sems.at[t, b],
                device_id=(j,),
                device_id_type=pl.DeviceIdType.MESH,
            )

        @pl.when(my_pos == 0)
        def _():
            for (t, b) in CHUNKS:
                cps[(t, b)].wait()
                flat = jnp.reshape(kv_f32[t, b].astype(BF16),
                                   (SKV, HQ_ALL * DH))
                kv_recv[t, b] = flat[:, 0:D_HEADS]
                for j in range(1, N_DEV):
                    kv_send[j - 1, t, b] = flat[:, j * D_HEADS:(j + 1) * D_HEADS]
                for j in range(1, N_DEV):
                    kv_scatter_rdma(j, t, b).start()

        qb = lax.broadcasted_iota(jnp.int32, (SQ, SKV), 0) // BLK
        kb = lax.broadcasted_iota(jnp.int32, (SQ, SKV), 1) // BLK
        mask = kb <= qb

        acc = []
        ar_rdmas = []
        for b in range(B):
            @pl.when(my_pos != 0)
            def _(b=b):
                kv_scatter_rdma(1, 0, b).wait_recv()
                kv_scatter_rdma(1, 1, b).wait_recv()

            acc_b = None
            for hl in range(HQ):
                c = slice(hl * DH, (hl + 1) * DH)
                qh = q[b][:, c]
                kh = kv_recv[0, b, :, c]
                vh = kv_recv[1, b, :, c]
                s = lax.dot_general(qh, kh, (((1,), (1,)), ((), ())),
                                    preferred_element_type=F32)
                s = jnp.where(mask, s * 0.125, -1e9)
                m = jnp.max(s, axis=1, keepdims=True)
                w = jnp.exp(s - m)
                w = w / jnp.sum(w, axis=1, keepdims=True)
                ctx = lax.dot(w.astype(BF16), vh,
                              preferred_element_type=F32)
                contrib = lax.dot(ctx.astype(BF16), wo[c, :],
                                  preferred_element_type=F32)
                acc_b = contrib if acc_b is None else acc_b + contrib
            acc.append(acc_b)
            ar_stage[b] = acc_b.astype(BF16)
            for off in range(1, N_DEV):
                dest = (my_pos + off) % N_DEV
                rdma = pltpu.make_async_remote_copy(
                    src_ref=ar_stage.at[b],
                    dst_ref=ar_recv.at[off - 1, b],
                    send_sem=ar_send_sems.at[off - 1, b],
                    recv_sem=ar_recv_sems.at[off - 1, b],
                    device_id=(dest,),
                    device_id_type=pl.DeviceIdType.MESH,
                )
                rdma.start()
                ar_rdmas.append(rdma)

        @pl.when(my_pos == 0)
        def _():
            for (t, b) in CHUNKS:
                for j in range(1, N_DEV):
                    kv_scatter_rdma(j, t, b).wait_send()

        for rdma in ar_rdmas:
            rdma.wait()

        for b in range(B):
            total = acc[b]
            for off in range(1, N_DEV):
                total = total + ar_recv[off - 1, b].astype(F32)
            out_ref[b] = total

    return pl.pallas_call(
        body,
        out_shape=jax.ShapeDtypeStruct((B, SQ, D_MODEL), F32),
        in_specs=[
            pl.BlockSpec(memory_space=pltpu.VMEM),
            pl.BlockSpec(memory_space=pltpu.VMEM),
            pl.BlockSpec(memory_space=pl.ANY),
            pl.BlockSpec(memory_space=pl.ANY),
            pl.BlockSpec(memory_space=pltpu.VMEM),
        ],
        out_specs=pl.BlockSpec(memory_space=pltpu.VMEM),
        scratch_shapes=[
            pltpu.VMEM((2, B, SKV, HQ_ALL, DH), F32),
            pltpu.VMEM((N_DEV - 1, 2, B, SKV, D_HEADS), BF16),
            pltpu.VMEM((2, B, SKV, D_HEADS), BF16),
            pltpu.VMEM((B, SQ, D_MODEL), BF16),
            pltpu.VMEM((N_DEV - 1, B, SQ, D_MODEL), BF16),
            pltpu.SemaphoreType.DMA((2, B)),
            pltpu.SemaphoreType.DMA((N_DEV - 1, 2, B)),
            pltpu.SemaphoreType.DMA((2, B)),
            pltpu.SemaphoreType.DMA((N_DEV - 1, B)),
            pltpu.SemaphoreType.DMA((N_DEV - 1, B)),
        ],
        compiler_params=pltpu.CompilerParams(collective_id=0),
    )(x, Wq, K_ext, V_ext, Wo)


# device time: 17393 ns/iter; 1.6607x vs baseline; 1.2212x over previous
import jax
import jax.numpy as jnp
from jax import lax
from jax.experimental import pallas as pl
from jax.experimental.pallas import tpu as pltpu

N_DEV = 4
B = 2
SQ = 128
SKV = 128
HQ = 4
DH = 64
HQ_ALL = 16
D_HEADS = HQ * DH
D_MODEL = 512
BLK = 64
BF16 = jnp.bfloat16
F32 = jnp.float32

CHUNKS = [(0, 0), (1, 0), (0, 1), (1, 1)]


def kernel(x, Wq, K_ext, V_ext, Wo):
    def body(x_ref, wq_ref, k_ref, v_ref, wo_ref, out_ref,
             kv_f32, kv_send, kv_recv, ar_stage, ar_recv,
             local_sems, kv_send_sems, kv_recv_sems, ar_send_sems, ar_recv_sems):
        my_pos = lax.axis_index("i")

        tv_ref = [k_ref, v_ref]
        cps = {
            (t, b): pltpu.make_async_copy(
                tv_ref[t].at[b], kv_f32.at[t, b], local_sems.at[t, b])
            for (t, b) in CHUNKS
        }

        @pl.when(my_pos == 0)
        def _():
            for tb in CHUNKS:
                cps[tb].start()

        barrier_sem = pltpu.get_barrier_semaphore()
        for off in range(1, N_DEV):
            peer = (my_pos + off) % N_DEV
            pl.semaphore_signal(barrier_sem, inc=1, device_id=(peer,),
                                device_id_type=pl.DeviceIdType.MESH)
        pl.semaphore_wait(barrier_sem, N_DEV - 1)

        wq = wq_ref[:].astype(BF16)
        wo = wo_ref[:].astype(BF16)
        q = []
        for b in range(B):
            xb = x_ref[b].astype(BF16)
            q.append(lax.dot(xb, wq,
                             preferred_element_type=F32).astype(BF16))

        def kv_scatter_rdma(j, t, b):
            return pltpu.make_async_remote_copy(
                src_ref=kv_send.at[j - 1, t, b],
                dst_ref=kv_recv.at[t, b],
                send_sem=kv_send_sems.at[j - 1, t, b],
                recv_sem=kv_recv_sems.at[t, b],
                device_id=(j,),
                device_id_type=pl.DeviceIdType.MESH,
            )

        @pl.when(my_pos == 0)
        def _():
            for (t, b) in CHUNKS:
                cps[(t, b)].wait()
                flat = jnp.reshape(kv_f32[t, b].astype(BF16),
                                   (SKV, HQ_ALL * DH))
                kv_recv[t, b] = flat[:, 0:D_HEADS]
                for j in range(1, N_DEV):
                    kv_send[j - 1, t, b] = flat[:, j * D_HEADS:(j + 1) * D_HEADS]
                for j in range(1, N_DEV):
                    kv_scatter_rdma(j, t, b).start()

        row = lax.broadcasted_iota(jnp.int32, (HQ * SQ, SKV), 0)
        qb = (row % SQ) // BLK
        kb = lax.broadcasted_iota(jnp.int32, (HQ * SQ, SKV), 1) // BLK
        mask = kb <= qb

        acc = []
        ar_rdmas = {}
        for b in range(B):
            @pl.when(my_pos != 0)
            def _(b=b):
                kv_scatter_rdma(1, 0, b).wait_recv()
                kv_scatter_rdma(1, 1, b).wait_recv()

            s_list = []
            for hl in range(HQ):
                c = slice(hl * DH, (hl + 1) * DH)
                s_list.append(
                    lax.dot_general(q[b][:, c], kv_recv[0, b, :, c],
                                    (((1,), (1,)), ((), ())),
                                    preferred_element_type=F32))
            s = jnp.concatenate(s_list, axis=0)
            w = jnp.exp(jnp.where(mask, s * 0.125, -1e9))
            w = (w / jnp.sum(w, axis=1, keepdims=True)).astype(BF16)
            ctx_list = []
            for hl in range(HQ):
                c = slice(hl * DH, (hl + 1) * DH)
                ctx_list.append(
                    lax.dot(w[hl * SQ:(hl + 1) * SQ, :], kv_recv[1, b, :, c],
                            preferred_element_type=F32).astype(BF16))
            ctx = jnp.concatenate(ctx_list, axis=1)
            acc_b = lax.dot(ctx, wo, preferred_element_type=F32)
            acc.append(acc_b)
            ar_stage[b] = acc_b.astype(BF16)
            for off in range(1, N_DEV):
                dest = (my_pos + off) % N_DEV
                rdma = pltpu.make_async_remote_copy(
                    src_ref=ar_stage.at[b],
                    dst_ref=ar_recv.at[off - 1, b],
                    send_sem=ar_send_sems.at[off - 1, b],
                    recv_sem=ar_recv_sems.at[off - 1, b],
                    device_id=(dest,),
                    device_id_type=pl.DeviceIdType.MESH,
                )
                rdma.start()
                ar_rdmas[(off, b)] = rdma

        @pl.when(my_pos == 0)
        def _():
            for (t, b) in CHUNKS:
                for j in range(1, N_DEV):
                    kv_scatter_rdma(j, t, b).wait_send()

        for b in range(B):
            total = acc[b]
            for off in range(1, N_DEV):
                ar_rdmas[(off, b)].wait_recv()
                total = total + ar_recv[off - 1, b].astype(F32)
            out_ref[b] = total

        for rdma in ar_rdmas.values():
            rdma.wait_send()

    return pl.pallas_call(
        body,
        out_shape=jax.ShapeDtypeStruct((B, SQ, D_MODEL), F32),
        in_specs=[
            pl.BlockSpec(memory_space=pltpu.VMEM),
            pl.BlockSpec(memory_space=pltpu.VMEM),
            pl.BlockSpec(memory_space=pl.ANY),
            pl.BlockSpec(memory_space=pl.ANY),
            pl.BlockSpec(memory_space=pltpu.VMEM),
        ],
        out_specs=pl.BlockSpec(memory_space=pltpu.VMEM),
        scratch_shapes=[
            pltpu.VMEM((2, B, SKV, HQ_ALL, DH), F32),
            pltpu.VMEM((N_DEV - 1, 2, B, SKV, D_HEADS), BF16),
            pltpu.VMEM((2, B, SKV, D_HEADS), BF16),
            pltpu.VMEM((B, SQ, D_MODEL), BF16),
            pltpu.VMEM((N_DEV - 1, B, SQ, D_MODEL), BF16),
            pltpu.SemaphoreType.DMA((2, B)),
            pltpu.SemaphoreType.DMA((N_DEV - 1, 2, B)),
            pltpu.SemaphoreType.DMA((2, B)),
            pltpu.SemaphoreType.DMA((N_DEV - 1, B)),
            pltpu.SemaphoreType.DMA((N_DEV - 1, B)),
        ],
        compiler_params=pltpu.CompilerParams(collective_id=0),
    )(x, Wq, K_ext, V_ext, Wo)
